# baseline (speedup 1.0000x reference)
import functools

import jax
import jax.numpy as jnp
from jax.experimental import pallas as pl


N, FEA, CLA, K = 8192, 128, 8, 40
S_DIM, P_DIM = 4, 22


def _mlp_kernel(cat_ref, wf1_ref, bf1_ref, wf2_ref, bf2_ref, out_ref):
    cat = cat_ref[...]
    hid = jnp.maximum(
        jnp.dot(cat, wf1_ref[...], preferred_element_type=jnp.float32)
        + bf1_ref[...], 0.0)
    out_ref[...] = (
        jnp.dot(hid, wf2_ref[...], preferred_element_type=jnp.float32)
        + bf2_ref[...])


def _final_mlp(cat, Wf1, bf1, Wf2, bf2):
    R = 512
    grid = (cat.shape[0] // R,)
    return pl.pallas_call(
        _mlp_kernel,
        grid=grid,
        in_specs=[
            pl.BlockSpec((R, cat.shape[1]), lambda i: (i, 0)),
            pl.BlockSpec(Wf1.shape, lambda i: (0, 0)),
            pl.BlockSpec((1, 64), lambda i: (0, 0)),
            pl.BlockSpec(Wf2.shape, lambda i: (0, 0)),
            pl.BlockSpec((1, CLA), lambda i: (0, 0)),
        ],
        out_specs=pl.BlockSpec((R, CLA), lambda i: (i, 0)),
        out_shape=jax.ShapeDtypeStruct((cat.shape[0], CLA), jnp.float32),
    )(cat, Wf1, bf1.reshape(1, -1), Wf2, bf2.reshape(1, -1))


def _gravnet_conv(x, p):
    s = x @ p['Ws'] + p['bs']
    h = x @ p['Wh'] + p['bh']
    sq = jnp.sum(s * s, axis=1)
    d2 = sq[:, None] + sq[None, :] - 2.0 * (s @ s.T)
    _, idx = jax.lax.top_k(-d2, K)
    diff = s[:, None, :] - s[idx]
    w = jnp.exp(-10.0 * jnp.sum(diff * diff, axis=-1))
    msg = h[idx] * w[..., None]
    agg = jnp.concatenate([jnp.mean(msg, axis=1), jnp.max(msg, axis=1)], axis=-1)
    return x @ p['Wo1'] + agg @ p['Wo2'] + p['bo2']


def kernel(x, batch, params):
    x1 = _gravnet_conv(x, params['conv1'])
    x2 = _gravnet_conv(x1, params['conv2'])
    x3 = _gravnet_conv(x2, params['conv3'])
    x4 = _gravnet_conv(x3, params['conv4'])
    cat = jnp.concatenate([x1, x2, x3, x4], axis=1)
    return _final_mlp(cat, params['Wf1'], params['bf1'],
                      params['Wf2'], params['bf2'])


# speedup vs baseline: 19.3636x; 19.3636x over previous
import functools

import jax
import jax.numpy as jnp
from jax import lax
from jax.experimental import pallas as pl
from jax.experimental.pallas import tpu as pltpu
from jax.experimental.pallas import tpu_sc as plsc

N, FEA, CLA, KNN = 8192, 128, 8, 40
S_DIM, P_DIM = 4, 22
TBL = 128
NBUCK = 128
DEPTH = 8
IMAX = 2**31 - 1


def _proj_body(x_ref, s_ref, h_ref, wo1_ref,
               aug_ref, tbl_ref, xwo1_ref):
    x = x_ref[...]
    s = s_ref[...]
    h = h_ref[...]
    sq = jnp.sum(s * s, axis=1, keepdims=True)
    one = jnp.ones_like(sq)
    r = x.shape[0]
    aug_ref[...] = jnp.concatenate(
        [s, one, sq, jnp.zeros((r, 2), jnp.float32)], axis=1)
    tbl_ref[...] = jnp.concatenate(
        [s, jnp.zeros((r, 12), jnp.float32),
         h, jnp.zeros((r, TBL - 16 - P_DIM), jnp.float32)], axis=1)
    xwo1_ref[...] = jnp.dot(x, wo1_ref[...], preferred_element_type=jnp.float32)


def _proj(x, p):
    din = x.shape[1]
    dout = p['Wo1'].shape[1]
    s = x @ p['Ws'] + p['bs']
    h = x @ p['Wh'] + p['bh']
    R = 512
    out = pl.pallas_call(
        _proj_body,
        grid=(N // R,),
        in_specs=[
            pl.BlockSpec((R, din), lambda i: (i, 0)),
            pl.BlockSpec((R, S_DIM), lambda i: (i, 0)),
            pl.BlockSpec((R, P_DIM), lambda i: (i, 0)),
            pl.BlockSpec((din, dout), lambda i: (0, 0)),
        ],
        out_specs=[
            pl.BlockSpec((R, 8), lambda i: (i, 0)),
            pl.BlockSpec((R, TBL), lambda i: (i, 0)),
            pl.BlockSpec((R, dout), lambda i: (i, 0)),
        ],
        out_shape=[
            jax.ShapeDtypeStruct((N, 8), jnp.float32),
            jax.ShapeDtypeStruct((N, TBL), jnp.float32),
            jax.ShapeDtypeStruct((N, dout), jnp.float32),
        ],
    )(x, s, h, p['Wo1'])
    return out


def _topk_body(aug_ref, augt_ref, idx_ref):
    ab = aug_ref[...]
    r = ab.shape[0]
    s_blk = ab[:, 0:4]
    sq_blk = ab[:, 5:6]
    lane = lax.broadcasted_iota(jnp.int32, (r, NBUCK), 1)
    m = [jnp.full((r, NBUCK), IMAX, jnp.int32) for _ in range(DEPTH)]
    for c in range(N // NBUCK):
        st_c = augt_ref[0:4, c * NBUCK:(c + 1) * NBUCK]
        sq_c = augt_ref[5:6, c * NBUCK:(c + 1) * NBUCK]
        prod = jnp.dot(s_blk, st_c, preferred_element_type=jnp.float32)
        d = (sq_blk + sq_c) - 2.0 * prod
        d = jnp.maximum(d, 0.0)
        bits = lax.bitcast_convert_type(d, jnp.int32)
        x = (bits & jnp.int32(-64)) | jnp.int32(c)
        for t in range(DEPTH):
            lo = jnp.minimum(m[t], x)
            x = jnp.maximum(m[t], x)
            m[t] = lo
    outs = []
    for _ in range(KNN):
        tmin = jnp.min(m[0], axis=1, keepdims=True)
        win = m[0] == tmin
        lane_sel = jnp.where(win, lane, jnp.int32(N))
        min_lane = jnp.min(lane_sel, axis=1, keepdims=True)
        win1 = win & (lane == min_lane)
        outs.append((tmin & jnp.int32(63)) * NBUCK + min_lane)
        for t in range(DEPTH - 1):
            m[t] = jnp.where(win1, m[t + 1], m[t])
        m[DEPTH - 1] = jnp.where(win1, IMAX, m[DEPTH - 1])
    idx_ref[...] = jnp.concatenate(outs, axis=1)


def _topk(aug, augt):
    R = 256
    return pl.pallas_call(
        _topk_body,
        grid=(N // R,),
        in_specs=[
            pl.BlockSpec((R, 8), lambda i: (i, 0)),
            pl.BlockSpec((8, N), lambda i: (0, 0)),
        ],
        out_specs=pl.BlockSpec((R, KNN), lambda i: (i, 0)),
        out_shape=jax.ShapeDtypeStruct((N, KNN), jnp.int32),
    )(aug, augt)


_SC_NW = 32
_CH = 512


def _make_sc_gather():
    mesh = plsc.VectorSubcoreMesh(core_axis_name="c", subcore_axis_name="s")

    @functools.partial(
        pl.kernel, mesh=mesh,
        out_type=jax.ShapeDtypeStruct((N * KNN, TBL), jnp.float32),
        scratch_types=[
            pltpu.VMEM((_CH,), jnp.int32),
            pltpu.VMEM((_CH, TBL), jnp.float32),
            pltpu.SemaphoreType.DMA,
        ],
    )
    def gather_k(table_hbm, idx_hbm, out_hbm, idx_v, rows_v, sem):
        wid = lax.axis_index("s") * 2 + lax.axis_index("c")
        b_per_w = (N * KNN) // _SC_NW
        base = wid * b_per_w
        for t in range(b_per_w // _CH):
            off = base + t * _CH
            pltpu.sync_copy(idx_hbm.at[pl.ds(off, _CH)], idx_v)
            pltpu.async_copy(table_hbm.at[idx_v], rows_v, sem).wait()
            pltpu.sync_copy(rows_v, out_hbm.at[pl.ds(off, _CH)])

    return gather_k


_sc_gather_fn = None


def _sc_gather(table, idxflat):
    global _sc_gather_fn
    if _sc_gather_fn is None:
        _sc_gather_fn = _make_sc_gather()
    return _sc_gather_fn(table, idxflat)


def _agg_body(g_ref, tbl_ref, xwo1_ref, wo2_ref, bo2_ref, out_ref):
    g = g_ref[...]
    s = tbl_ref[:, 0:S_DIM]
    diff = g[:, :, 0:S_DIM] - s[:, None, :]
    d2 = jnp.sum(diff * diff, axis=2)
    w = jnp.exp(-10.0 * d2)
    msg = g[:, :, 16:16 + P_DIM] * w[:, :, None]
    mean = jnp.sum(msg, axis=1) * (1.0 / KNN)
    mx = jnp.max(msg, axis=1)
    agg = jnp.concatenate([mean, mx], axis=1)
    out_ref[...] = (xwo1_ref[...]
                    + jnp.dot(agg, wo2_ref[...],
                              preferred_element_type=jnp.float32)
                    + bo2_ref[...])


def _agg(g3, table, xwo1, Wo2, bo2):
    dout = Wo2.shape[1]
    R = 256
    return pl.pallas_call(
        _agg_body,
        grid=(N // R,),
        in_specs=[
            pl.BlockSpec((R, KNN, TBL), lambda i: (i, 0, 0)),
            pl.BlockSpec((R, TBL), lambda i: (i, 0)),
            pl.BlockSpec((R, dout), lambda i: (i, 0)),
            pl.BlockSpec((2 * P_DIM, dout), lambda i: (0, 0)),
            pl.BlockSpec((1, dout), lambda i: (0, 0)),
        ],
        out_specs=pl.BlockSpec((R, dout), lambda i: (i, 0)),
        out_shape=jax.ShapeDtypeStruct((N, dout), jnp.float32),
    )(g3, table, xwo1, Wo2, bo2.reshape(1, -1))


def _mlp_body(cat_ref, wf1_ref, bf1_ref, wf2_ref, bf2_ref, out_ref):
    hid = jnp.maximum(
        jnp.dot(cat_ref[...], wf1_ref[...],
                preferred_element_type=jnp.float32) + bf1_ref[...], 0.0)
    out_ref[...] = (jnp.dot(hid, wf2_ref[...],
                            preferred_element_type=jnp.float32)
                    + bf2_ref[...])


def _final_mlp(cat, Wf1, bf1, Wf2, bf2):
    R = 512
    dcat = cat.shape[1]
    return pl.pallas_call(
        _mlp_body,
        grid=(N // R,),
        in_specs=[
            pl.BlockSpec((R, dcat), lambda i: (i, 0)),
            pl.BlockSpec(Wf1.shape, lambda i: (0, 0)),
            pl.BlockSpec((1, 64), lambda i: (0, 0)),
            pl.BlockSpec(Wf2.shape, lambda i: (0, 0)),
            pl.BlockSpec((1, CLA), lambda i: (0, 0)),
        ],
        out_specs=pl.BlockSpec((R, CLA), lambda i: (i, 0)),
        out_shape=jax.ShapeDtypeStruct((N, CLA), jnp.float32),
    )(cat, Wf1, bf1.reshape(1, -1), Wf2, bf2.reshape(1, -1))


def _layer(x, p):
    aug, table, xwo1 = _proj(x, p)
    idx = _topk(aug, aug.T)
    g = _sc_gather(table, idx.reshape(-1))
    g3 = g.reshape(N, KNN, TBL)
    return _agg(g3, table, xwo1, p['Wo2'], p['bo2'])


def kernel(x, batch, params):
    x1 = _layer(x, params['conv1'])
    x2 = _layer(x1, params['conv2'])
    x3 = _layer(x2, params['conv3'])
    x4 = _layer(x3, params['conv4'])
    cat = jnp.concatenate([x1, x2, x3, x4], axis=1)
    return _final_mlp(cat, params['Wf1'], params['bf1'],
                      params['Wf2'], params['bf2'])


# speedup vs baseline: 21.0993x; 1.0896x over previous
import functools

import jax
import jax.numpy as jnp
from jax import lax
from jax.experimental import pallas as pl
from jax.experimental.pallas import tpu as pltpu
from jax.experimental.pallas import tpu_sc as plsc

N, FEA, CLA, KNN = 8192, 128, 8, 40
S_DIM, P_DIM = 4, 22
TBL = 128
NBUCK = 128
DEPTH = 6
IMAX = 2**31 - 1


def _proj_common(x, ws, bs, wh, bh, wo1, aug_ref, tbl_ref, xwo1_ref):
    s = jnp.dot(x, ws, preferred_element_type=jnp.float32) + bs
    h = jnp.dot(x, wh, preferred_element_type=jnp.float32) + bh
    sq = jnp.sum(s * s, axis=1, keepdims=True)
    one = jnp.ones_like(sq)
    r = x.shape[0]
    aug_ref[...] = jnp.concatenate(
        [s, one, sq, jnp.zeros((r, 2), jnp.float32)], axis=1)
    tbl_ref[...] = jnp.concatenate(
        [s, jnp.zeros((r, 12), jnp.float32),
         h, jnp.zeros((r, TBL - 16 - P_DIM), jnp.float32)], axis=1)
    xwo1_ref[...] = jnp.dot(x, wo1, preferred_element_type=jnp.float32)


def _proj_body(x_ref, ws_ref, bs_ref, wh_ref, bh_ref, wo1_ref,
               aug_ref, tbl_ref, xwo1_ref):
    _proj_common(x_ref[...], ws_ref[...], bs_ref[...], wh_ref[...],
                 bh_ref[...], wo1_ref[...], aug_ref, tbl_ref, xwo1_ref)


def _proj(x, p):
    din = x.shape[1]
    dout = p['Wo1'].shape[1]
    R = 512
    out = pl.pallas_call(
        _proj_body,
        grid=(N // R,),
        in_specs=[
            pl.BlockSpec((R, din), lambda i: (i, 0)),
            pl.BlockSpec((din, S_DIM), lambda i: (0, 0)),
            pl.BlockSpec((1, S_DIM), lambda i: (0, 0)),
            pl.BlockSpec((din, P_DIM), lambda i: (0, 0)),
            pl.BlockSpec((1, P_DIM), lambda i: (0, 0)),
            pl.BlockSpec((din, dout), lambda i: (0, 0)),
        ],
        out_specs=[
            pl.BlockSpec((R, 8), lambda i: (i, 0)),
            pl.BlockSpec((R, TBL), lambda i: (i, 0)),
            pl.BlockSpec((R, dout), lambda i: (i, 0)),
        ],
        out_shape=[
            jax.ShapeDtypeStruct((N, 8), jnp.float32),
            jax.ShapeDtypeStruct((N, TBL), jnp.float32),
            jax.ShapeDtypeStruct((N, dout), jnp.float32),
        ],
    )(x, p['Ws'], p['bs'].reshape(1, -1), p['Wh'], p['bh'].reshape(1, -1),
      p['Wo1'])
    return out


def _topk_body(aug_ref, augt_ref, idx_ref):
    ab = aug_ref[...]
    r = ab.shape[0]
    s_blk = ab[:, 0:4]
    sq_blk = ab[:, 5:6]
    lane = lax.broadcasted_iota(jnp.int32, (r, NBUCK), 1)
    m = [jnp.full((r, NBUCK), IMAX, jnp.int32) for _ in range(DEPTH)]
    for c in range(N // NBUCK):
        st_c = augt_ref[0:4, c * NBUCK:(c + 1) * NBUCK]
        sq_c = augt_ref[5:6, c * NBUCK:(c + 1) * NBUCK]
        prod = jnp.dot(s_blk, st_c, preferred_element_type=jnp.float32)
        d = (sq_blk + sq_c) - 2.0 * prod
        d = jnp.maximum(d, 0.0)
        bits = lax.bitcast_convert_type(d, jnp.int32)
        x = (bits & jnp.int32(-64)) | jnp.int32(c)
        for t in range(DEPTH):
            lo = jnp.minimum(m[t], x)
            x = jnp.maximum(m[t], x)
            m[t] = lo
    outs = []
    for _ in range(KNN):
        tmin = jnp.min(m[0], axis=1, keepdims=True)
        win = m[0] == tmin
        lane_sel = jnp.where(win, lane, jnp.int32(N))
        min_lane = jnp.min(lane_sel, axis=1, keepdims=True)
        win1 = win & (lane == min_lane)
        outs.append((tmin & jnp.int32(63)) * NBUCK + min_lane)
        for t in range(DEPTH - 1):
            m[t] = jnp.where(win1, m[t + 1], m[t])
        m[DEPTH - 1] = jnp.where(win1, IMAX, m[DEPTH - 1])
    idx_ref[...] = jnp.concatenate(outs, axis=1)


def _topk(aug, augt):
    R = 256
    return pl.pallas_call(
        _topk_body,
        grid=(N // R,),
        in_specs=[
            pl.BlockSpec((R, 8), lambda i: (i, 0)),
            pl.BlockSpec((8, N), lambda i: (0, 0)),
        ],
        out_specs=pl.BlockSpec((R, KNN), lambda i: (i, 0)),
        out_shape=jax.ShapeDtypeStruct((N, KNN), jnp.int32),
    )(aug, augt)


_SC_NW = 32
_CH = 512


def _make_sc_gather():
    mesh = plsc.VectorSubcoreMesh(core_axis_name="c", subcore_axis_name="s")

    @functools.partial(
        pl.kernel, mesh=mesh,
        out_type=jax.ShapeDtypeStruct((N * KNN, TBL), jnp.float32),
        scratch_types=[
            pltpu.VMEM((_CH,), jnp.int32),
            pltpu.VMEM((_CH, TBL), jnp.float32),
            pltpu.SemaphoreType.DMA,
        ],
    )
    def gather_k(table_hbm, idx_hbm, out_hbm, idx_v, rows_v, sem):
        wid = lax.axis_index("s") * 2 + lax.axis_index("c")
        b_per_w = (N * KNN) // _SC_NW
        base = wid * b_per_w
        for t in range(b_per_w // _CH):
            off = base + t * _CH
            pltpu.sync_copy(idx_hbm.at[pl.ds(off, _CH)], idx_v)
            pltpu.async_copy(table_hbm.at[idx_v], rows_v, sem).wait()
            pltpu.sync_copy(rows_v, out_hbm.at[pl.ds(off, _CH)])

    return gather_k


_sc_gather_fn = None


def _sc_gather(table, idxflat):
    global _sc_gather_fn
    if _sc_gather_fn is None:
        _sc_gather_fn = _make_sc_gather()
    return _sc_gather_fn(table, idxflat)


def _agg_common(g, tbl, xwo1, wo2, bo2):
    s = tbl[:, 0:S_DIM]
    diff = g[:, :, 0:S_DIM] - s[:, None, :]
    d2 = jnp.sum(diff * diff, axis=2)
    w = jnp.exp(-10.0 * d2)
    msg = g[:, :, 16:16 + P_DIM] * w[:, :, None]
    mean = jnp.sum(msg, axis=1) * (1.0 / KNN)
    mx = jnp.max(msg, axis=1)
    agg = jnp.concatenate([mean, mx], axis=1)
    return (xwo1 + jnp.dot(agg, wo2, preferred_element_type=jnp.float32)
            + bo2)


def _agg_body(g_ref, tbl_ref, xwo1_ref, wo2_ref, bo2_ref, out_ref):
    out_ref[...] = _agg_common(g_ref[...], tbl_ref[...], xwo1_ref[...],
                               wo2_ref[...], bo2_ref[...])


def _aggproj_body(g_ref, tbl_ref, xwo1_ref, wo2_ref, bo2_ref,
                  ws2_ref, bs2_ref, wh2_ref, bh2_ref, wo12_ref,
                  out_ref, aug2_ref, tbl2_ref, xwo12_ref):
    out = _agg_common(g_ref[...], tbl_ref[...], xwo1_ref[...],
                      wo2_ref[...], bo2_ref[...])
    out_ref[...] = out
    _proj_common(out, ws2_ref[...], bs2_ref[...], wh2_ref[...],
                 bh2_ref[...], wo12_ref[...], aug2_ref, tbl2_ref, xwo12_ref)


def _aggproj(g3, table, xwo1, p, p2):
    dout = p['Wo2'].shape[1]
    dout2 = p2['Wo1'].shape[1]
    R = 256
    return pl.pallas_call(
        _aggproj_body,
        grid=(N // R,),
        in_specs=[
            pl.BlockSpec((R, KNN, TBL), lambda i: (i, 0, 0)),
            pl.BlockSpec((R, TBL), lambda i: (i, 0)),
            pl.BlockSpec((R, dout), lambda i: (i, 0)),
            pl.BlockSpec((2 * P_DIM, dout), lambda i: (0, 0)),
            pl.BlockSpec((1, dout), lambda i: (0, 0)),
            pl.BlockSpec((dout, S_DIM), lambda i: (0, 0)),
            pl.BlockSpec((1, S_DIM), lambda i: (0, 0)),
            pl.BlockSpec((dout, P_DIM), lambda i: (0, 0)),
            pl.BlockSpec((1, P_DIM), lambda i: (0, 0)),
            pl.BlockSpec((dout, dout2), lambda i: (0, 0)),
        ],
        out_specs=[
            pl.BlockSpec((R, dout), lambda i: (i, 0)),
            pl.BlockSpec((R, 8), lambda i: (i, 0)),
            pl.BlockSpec((R, TBL), lambda i: (i, 0)),
            pl.BlockSpec((R, dout2), lambda i: (i, 0)),
        ],
        out_shape=[
            jax.ShapeDtypeStruct((N, dout), jnp.float32),
            jax.ShapeDtypeStruct((N, 8), jnp.float32),
            jax.ShapeDtypeStruct((N, TBL), jnp.float32),
            jax.ShapeDtypeStruct((N, dout2), jnp.float32),
        ],
    )(g3, table, xwo1, p['Wo2'], p['bo2'].reshape(1, -1),
      p2['Ws'], p2['bs'].reshape(1, -1), p2['Wh'], p2['bh'].reshape(1, -1),
      p2['Wo1'])


def _agg(g3, table, xwo1, Wo2, bo2):
    dout = Wo2.shape[1]
    R = 256
    return pl.pallas_call(
        _agg_body,
        grid=(N // R,),
        in_specs=[
            pl.BlockSpec((R, KNN, TBL), lambda i: (i, 0, 0)),
            pl.BlockSpec((R, TBL), lambda i: (i, 0)),
            pl.BlockSpec((R, dout), lambda i: (i, 0)),
            pl.BlockSpec((2 * P_DIM, dout), lambda i: (0, 0)),
            pl.BlockSpec((1, dout), lambda i: (0, 0)),
        ],
        out_specs=pl.BlockSpec((R, dout), lambda i: (i, 0)),
        out_shape=jax.ShapeDtypeStruct((N, dout), jnp.float32),
    )(g3, table, xwo1, Wo2, bo2.reshape(1, -1))


def _mlp_body(cat_ref, wf1_ref, bf1_ref, wf2_ref, bf2_ref, out_ref):
    hid = jnp.maximum(
        jnp.dot(cat_ref[...], wf1_ref[...],
                preferred_element_type=jnp.float32) + bf1_ref[...], 0.0)
    out_ref[...] = (jnp.dot(hid, wf2_ref[...],
                            preferred_element_type=jnp.float32)
                    + bf2_ref[...])


def _final_mlp(cat, Wf1, bf1, Wf2, bf2):
    R = 512
    dcat = cat.shape[1]
    return pl.pallas_call(
        _mlp_body,
        grid=(N // R,),
        in_specs=[
            pl.BlockSpec((R, dcat), lambda i: (i, 0)),
            pl.BlockSpec(Wf1.shape, lambda i: (0, 0)),
            pl.BlockSpec((1, 64), lambda i: (0, 0)),
            pl.BlockSpec(Wf2.shape, lambda i: (0, 0)),
            pl.BlockSpec((1, CLA), lambda i: (0, 0)),
        ],
        out_specs=pl.BlockSpec((R, CLA), lambda i: (i, 0)),
        out_shape=jax.ShapeDtypeStruct((N, CLA), jnp.float32),
    )(cat, Wf1, bf1.reshape(1, -1), Wf2, bf2.reshape(1, -1))


def kernel(x, batch, params):
    convs = [params['conv1'], params['conv2'], params['conv3'],
             params['conv4']]
    aug, table, xwo1 = _proj(x, convs[0])
    outs = []
    for li in range(4):
        idx = _topk(aug, aug.T)
        g = _sc_gather(table, idx.reshape(-1))
        g3 = g.reshape(N, KNN, TBL)
        if li < 3:
            out, aug, table, xwo1 = _aggproj(g3, table, xwo1,
                                             convs[li], convs[li + 1])
        else:
            out = _agg(g3, table, xwo1, convs[li]['Wo2'], convs[li]['bo2'])
        outs.append(out)
    cat = jnp.concatenate(outs, axis=1)
    return _final_mlp(cat, params['Wf1'], params['bf1'],
                      params['Wf2'], params['bf2'])


# speedup vs baseline: 21.3340x; 1.0111x over previous
import functools

import jax
import jax.numpy as jnp
from jax import lax
from jax.experimental import pallas as pl
from jax.experimental.pallas import tpu as pltpu
from jax.experimental.pallas import tpu_sc as plsc

N, FEA, CLA, KNN = 8192, 128, 8, 40
S_DIM, P_DIM = 4, 22
TBL = 128
GW = 128
NBUCK = 128
DEPTH = 6
IMAX = 2**31 - 1


def _proj_common(x, ws, bs, wh, bh, wo1, aug_ref, tbl_ref, xwo1_ref):
    s = jnp.dot(x, ws, preferred_element_type=jnp.float32) + bs
    h = jnp.dot(x, wh, preferred_element_type=jnp.float32) + bh
    sq = jnp.sum(s * s, axis=1, keepdims=True)
    one = jnp.ones_like(sq)
    r = x.shape[0]
    aug_ref[...] = jnp.concatenate(
        [s, one, sq, jnp.zeros((r, 2), jnp.float32)], axis=1)
    tbl_ref[...] = jnp.concatenate(
        [s, h, jnp.zeros((r, TBL - S_DIM - P_DIM), jnp.float32)], axis=1)
    xwo1_ref[...] = jnp.dot(x, wo1, preferred_element_type=jnp.float32)


def _proj_body(x_ref, ws_ref, bs_ref, wh_ref, bh_ref, wo1_ref,
               aug_ref, tbl_ref, xwo1_ref):
    _proj_common(x_ref[...], ws_ref[...], bs_ref[...], wh_ref[...],
                 bh_ref[...], wo1_ref[...], aug_ref, tbl_ref, xwo1_ref)


def _proj(x, p):
    din = x.shape[1]
    dout = p['Wo1'].shape[1]
    R = 512
    out = pl.pallas_call(
        _proj_body,
        grid=(N // R,),
        in_specs=[
            pl.BlockSpec((R, din), lambda i: (i, 0)),
            pl.BlockSpec((din, S_DIM), lambda i: (0, 0)),
            pl.BlockSpec((1, S_DIM), lambda i: (0, 0)),
            pl.BlockSpec((din, P_DIM), lambda i: (0, 0)),
            pl.BlockSpec((1, P_DIM), lambda i: (0, 0)),
            pl.BlockSpec((din, dout), lambda i: (0, 0)),
        ],
        out_specs=[
            pl.BlockSpec((R, 8), lambda i: (i, 0)),
            pl.BlockSpec((R, TBL), lambda i: (i, 0)),
            pl.BlockSpec((R, dout), lambda i: (i, 0)),
        ],
        out_shape=[
            jax.ShapeDtypeStruct((N, 8), jnp.float32),
            jax.ShapeDtypeStruct((N, TBL), jnp.float32),
            jax.ShapeDtypeStruct((N, dout), jnp.float32),
        ],
    )(x, p['Ws'], p['bs'].reshape(1, -1), p['Wh'], p['bh'].reshape(1, -1),
      p['Wo1'])
    return out


def _topk_body(aug_ref, augt_ref, idx_ref):
    ab = aug_ref[...]
    r = ab.shape[0]
    s_blk = ab[:, 0:4]
    sq_blk = ab[:, 5:6]
    lane = lax.broadcasted_iota(jnp.int32, (r, NBUCK), 1)
    m = [jnp.full((r, NBUCK), IMAX, jnp.int32) for _ in range(DEPTH)]
    for c in range(N // NBUCK):
        st_c = augt_ref[0:4, c * NBUCK:(c + 1) * NBUCK]
        sq_c = augt_ref[5:6, c * NBUCK:(c + 1) * NBUCK]
        prod = jnp.dot(s_blk, st_c, preferred_element_type=jnp.float32)
        d = (sq_blk + sq_c) - 2.0 * prod
        d = jnp.maximum(d, 0.0)
        bits = lax.bitcast_convert_type(d, jnp.int32)
        x = (bits & jnp.int32(-64)) | jnp.int32(c)
        for t in range(DEPTH):
            lo = jnp.minimum(m[t], x)
            x = jnp.maximum(m[t], x)
            m[t] = lo
    outs = []
    for _ in range(KNN):
        tmin = jnp.min(m[0], axis=1, keepdims=True)
        win = m[0] == tmin
        lane_sel = jnp.where(win, lane, jnp.int32(N))
        min_lane = jnp.min(lane_sel, axis=1, keepdims=True)
        win1 = win & (lane == min_lane)
        outs.append((tmin & jnp.int32(63)) * NBUCK + min_lane)
        for t in range(DEPTH - 1):
            m[t] = jnp.where(win1, m[t + 1], m[t])
        m[DEPTH - 1] = jnp.where(win1, IMAX, m[DEPTH - 1])
    idx_ref[...] = jnp.concatenate(outs, axis=1)


def _topk(aug, augt):
    R = 256
    return pl.pallas_call(
        _topk_body,
        grid=(N // R,),
        in_specs=[
            pl.BlockSpec((R, 8), lambda i: (i, 0)),
            pl.BlockSpec((8, N), lambda i: (0, 0)),
        ],
        out_specs=pl.BlockSpec((R, KNN), lambda i: (i, 0)),
        out_shape=jax.ShapeDtypeStruct((N, KNN), jnp.int32),
    )(aug, augt)


_SC_NW = 32
_CH = 320


def _make_sc_gather():
    mesh = plsc.VectorSubcoreMesh(core_axis_name="c", subcore_axis_name="s")
    b_per_w = (N * KNN) // _SC_NW

    @functools.partial(
        pl.kernel, mesh=mesh,
        out_type=jax.ShapeDtypeStruct((N * KNN, GW), jnp.float32),
        scratch_types=[
            pltpu.VMEM((b_per_w,), jnp.int32),
            pltpu.VMEM((_CH, TBL), jnp.float32),
            pltpu.VMEM((_CH, TBL), jnp.float32),
            pltpu.SemaphoreType.DMA,
            pltpu.SemaphoreType.DMA,
        ],
    )
    def gather_k(table_hbm, idx_hbm, out_hbm, idx_v, rows_v0, rows_v1,
                 sem0, sem1):
        wid = lax.axis_index("s") * 2 + lax.axis_index("c")
        base = wid * b_per_w
        pltpu.sync_copy(idx_hbm.at[pl.ds(base, b_per_w)], idx_v)

        def body(t, carry):
            off = pl.multiple_of(t * (2 * _CH), 2 * _CH)
            c0 = pltpu.async_copy(
                table_hbm.at[idx_v.at[pl.ds(off, _CH)]], rows_v0, sem0)
            c1 = pltpu.async_copy(
                table_hbm.at[idx_v.at[pl.ds(off + _CH, _CH)]], rows_v1, sem1)
            c0.wait()
            pltpu.sync_copy(rows_v0.at[:, 0:GW],
                            out_hbm.at[pl.ds(base + off, _CH)])
            c1.wait()
            pltpu.sync_copy(rows_v1.at[:, 0:GW],
                            out_hbm.at[pl.ds(base + off + _CH, _CH)])
            return carry

        lax.fori_loop(0, b_per_w // (2 * _CH), body, 0)

    return gather_k


_sc_gather_fn = None


def _sc_gather(table, idxflat):
    global _sc_gather_fn
    if _sc_gather_fn is None:
        _sc_gather_fn = _make_sc_gather()
    return _sc_gather_fn(table, idxflat)


def _agg_common(g, tbl, xwo1, wo2, bo2):
    s = tbl[:, 0:S_DIM]
    diff = g[:, :, 0:S_DIM] - s[:, None, :]
    d2 = jnp.sum(diff * diff, axis=2)
    w = jnp.exp(-10.0 * d2)
    msg = g[:, :, S_DIM:S_DIM + P_DIM] * w[:, :, None]
    mean = jnp.sum(msg, axis=1) * (1.0 / KNN)
    mx = jnp.max(msg, axis=1)
    agg = jnp.concatenate([mean, mx], axis=1)
    return (xwo1 + jnp.dot(agg, wo2, preferred_element_type=jnp.float32)
            + bo2)


def _agg_body(g_ref, tbl_ref, xwo1_ref, wo2_ref, bo2_ref, out_ref):
    out_ref[...] = _agg_common(g_ref[...], tbl_ref[...], xwo1_ref[...],
                               wo2_ref[...], bo2_ref[...])


def _aggproj_body(g_ref, tbl_ref, xwo1_ref, wo2_ref, bo2_ref,
                  ws2_ref, bs2_ref, wh2_ref, bh2_ref, wo12_ref,
                  out_ref, aug2_ref, tbl2_ref, xwo12_ref):
    out = _agg_common(g_ref[...], tbl_ref[...], xwo1_ref[...],
                      wo2_ref[...], bo2_ref[...])
    out_ref[...] = out
    _proj_common(out, ws2_ref[...], bs2_ref[...], wh2_ref[...],
                 bh2_ref[...], wo12_ref[...], aug2_ref, tbl2_ref, xwo12_ref)


def _aggproj(g3, table, xwo1, p, p2):
    dout = p['Wo2'].shape[1]
    dout2 = p2['Wo1'].shape[1]
    R = 256
    return pl.pallas_call(
        _aggproj_body,
        grid=(N // R,),
        in_specs=[
            pl.BlockSpec((R, KNN, GW), lambda i: (i, 0, 0)),
            pl.BlockSpec((R, TBL), lambda i: (i, 0)),
            pl.BlockSpec((R, dout), lambda i: (i, 0)),
            pl.BlockSpec((2 * P_DIM, dout), lambda i: (0, 0)),
            pl.BlockSpec((1, dout), lambda i: (0, 0)),
            pl.BlockSpec((dout, S_DIM), lambda i: (0, 0)),
            pl.BlockSpec((1, S_DIM), lambda i: (0, 0)),
            pl.BlockSpec((dout, P_DIM), lambda i: (0, 0)),
            pl.BlockSpec((1, P_DIM), lambda i: (0, 0)),
            pl.BlockSpec((dout, dout2), lambda i: (0, 0)),
        ],
        out_specs=[
            pl.BlockSpec((R, dout), lambda i: (i, 0)),
            pl.BlockSpec((R, 8), lambda i: (i, 0)),
            pl.BlockSpec((R, TBL), lambda i: (i, 0)),
            pl.BlockSpec((R, dout2), lambda i: (i, 0)),
        ],
        out_shape=[
            jax.ShapeDtypeStruct((N, dout), jnp.float32),
            jax.ShapeDtypeStruct((N, 8), jnp.float32),
            jax.ShapeDtypeStruct((N, TBL), jnp.float32),
            jax.ShapeDtypeStruct((N, dout2), jnp.float32),
        ],
    )(g3, table, xwo1, p['Wo2'], p['bo2'].reshape(1, -1),
      p2['Ws'], p2['bs'].reshape(1, -1), p2['Wh'], p2['bh'].reshape(1, -1),
      p2['Wo1'])


def _agg(g3, table, xwo1, Wo2, bo2):
    dout = Wo2.shape[1]
    R = 256
    return pl.pallas_call(
        _agg_body,
        grid=(N // R,),
        in_specs=[
            pl.BlockSpec((R, KNN, GW), lambda i: (i, 0, 0)),
            pl.BlockSpec((R, TBL), lambda i: (i, 0)),
            pl.BlockSpec((R, dout), lambda i: (i, 0)),
            pl.BlockSpec((2 * P_DIM, dout), lambda i: (0, 0)),
            pl.BlockSpec((1, dout), lambda i: (0, 0)),
        ],
        out_specs=pl.BlockSpec((R, dout), lambda i: (i, 0)),
        out_shape=jax.ShapeDtypeStruct((N, dout), jnp.float32),
    )(g3, table, xwo1, Wo2, bo2.reshape(1, -1))


def _mlp_body(cat_ref, wf1_ref, bf1_ref, wf2_ref, bf2_ref, out_ref):
    hid = jnp.maximum(
        jnp.dot(cat_ref[...], wf1_ref[...],
                preferred_element_type=jnp.float32) + bf1_ref[...], 0.0)
    out_ref[...] = (jnp.dot(hid, wf2_ref[...],
                            preferred_element_type=jnp.float32)
                    + bf2_ref[...])


def _final_mlp(cat, Wf1, bf1, Wf2, bf2):
    R = 512
    dcat = cat.shape[1]
    return pl.pallas_call(
        _mlp_body,
        grid=(N // R,),
        in_specs=[
            pl.BlockSpec((R, dcat), lambda i: (i, 0)),
            pl.BlockSpec(Wf1.shape, lambda i: (0, 0)),
            pl.BlockSpec((1, 64), lambda i: (0, 0)),
            pl.BlockSpec(Wf2.shape, lambda i: (0, 0)),
            pl.BlockSpec((1, CLA), lambda i: (0, 0)),
        ],
        out_specs=pl.BlockSpec((R, CLA), lambda i: (i, 0)),
        out_shape=jax.ShapeDtypeStruct((N, CLA), jnp.float32),
    )(cat, Wf1, bf1.reshape(1, -1), Wf2, bf2.reshape(1, -1))


def kernel(x, batch, params):
    convs = [params['conv1'], params['conv2'], params['conv3'],
             params['conv4']]
    aug, table, xwo1 = _proj(x, convs[0])
    outs = []
    for li in range(4):
        idx = _topk(aug, aug.T)
        g = _sc_gather(table, idx.reshape(-1))
        g3 = g.reshape(N, KNN, GW)
        if li < 3:
            out, aug, table, xwo1 = _aggproj(g3, table, xwo1,
                                             convs[li], convs[li + 1])
        else:
            out = _agg(g3, table, xwo1, convs[li]['Wo2'], convs[li]['bo2'])
        outs.append(out)
    cat = jnp.concatenate(outs, axis=1)
    return _final_mlp(cat, params['Wf1'], params['bf1'],
                      params['Wf2'], params['bf2'])
